# baseline (speedup 1.0000x reference)
import functools

import jax
import jax.numpy as jnp
from jax import lax
from jax.experimental import pallas as pl
from jax.experimental.pallas import tpu as pltpu
from jax.experimental.pallas import tpu_sc as plsc

N_NODES = 10000
N_EDGES = 640000
EDGE_DIM = 16
HIDDEN = 128
NUM_CLASSES = 10
NUM_GRAPHS = 64

NC = 2
NS = 16
NW = NC * NS

CH = 128
BLK = CH
EPC = N_EDGES // NW
NB = EPC // BLK
TAIL = EPC - NB * BLK
N_ACC = 10112
STRIPE = N_ACC // NS

MLP_TILE = 2000
LOOP_TILE = 1264
POOL_TILE = 1264


def _sc_mesh():
    return plsc.VectorSubcoreMesh(core_axis_name="c", subcore_axis_name="s")


@functools.partial(
    pl.kernel,
    mesh=_sc_mesh(),
    out_type=jax.ShapeDtypeStruct((NC, N_ACC, HIDDEN), jnp.float32),
    scratch_types=[
        pltpu.VMEM((CH,), jnp.int32),
        pltpu.VMEM((CH,), jnp.int32),
        pltpu.VMEM((TAIL,), jnp.int32),
        pltpu.VMEM((BLK, EDGE_DIM), jnp.float32),
        pltpu.VMEM((BLK, EDGE_DIM), jnp.float32),
        pltpu.VMEM((BLK, HIDDEN), jnp.float32),
        pltpu.VMEM_SHARED((N_ACC, HIDDEN), jnp.float32),
        pltpu.SemaphoreType.DMA,
        pltpu.SemaphoreType.DMA,
        pltpu.SemaphoreType.DMA,
        pltpu.SemaphoreType.DMA,
    ],
)
def _sc_edge_stats(dst_hbm, ea_hbm, init_hbm, zs_hbm, s_out,
                   idx0, idx1, idxt, attr0, attr1, rows_v, s_acc,
                   si0, si1, sa0, sa1):
    c = lax.axis_index("c")
    s = lax.axis_index("s")
    row0 = s * STRIPE
    pltpu.sync_copy(zs_hbm, s_acc.at[pl.ds(row0, STRIPE)])
    pltpu.sync_copy(init_hbm, rows_v)

    base = (c * NS + s) * EPC

    def start(blk, idx_v, attr_v, sem_i, sem_a):
        off = pl.multiple_of(base + blk * BLK, 8)
        pltpu.make_async_copy(dst_hbm.at[pl.ds(off, CH)], idx_v, sem_i).start()
        pltpu.make_async_copy(ea_hbm.at[pl.ds(off, BLK)], attr_v, sem_a).start()

    def wait(idx_v, attr_v, sem_i, sem_a):
        pltpu.make_async_copy(dst_hbm.at[pl.ds(0, CH)], idx_v, sem_i).wait()
        pltpu.make_async_copy(ea_hbm.at[pl.ds(0, BLK)], attr_v, sem_a).wait()

    def process(idx_v, attr_v):
        def widen(i8, carry2):
            for u in range(8):
                i = i8 * 8 + u
                rows_v[i, pl.ds(0, EDGE_DIM)] = attr_v[i, :]
            return carry2

        lax.fori_loop(0, BLK // 8, widen, 0)
        pltpu.sync_copy(rows_v, s_acc.at[idx_v], add=True)

    start(0, idx0, attr0, si0, sa0)
    start(1, idx1, attr1, si1, sa1)
    plsc.subcore_barrier()

    def body(kk, carry):
        j0 = 2 * kk
        wait(idx0, attr0, si0, sa0)
        process(idx0, attr0)

        @pl.when(j0 + 2 < NB)
        def _():
            start(j0 + 2, idx0, attr0, si0, sa0)

        wait(idx1, attr1, si1, sa1)
        process(idx1, attr1)

        @pl.when(j0 + 3 < NB)
        def _():
            start(j0 + 3, idx1, attr1, si1, sa1)

        return carry

    lax.fori_loop(0, NB // 2, body, 0)

    toff = pl.multiple_of(base + NB * BLK, 8)
    pltpu.sync_copy(dst_hbm.at[pl.ds(toff, TAIL)], idxt)
    pltpu.sync_copy(ea_hbm.at[pl.ds(toff, TAIL)], attr0.at[pl.ds(0, TAIL)])

    def widen_t(i8, carry2):
        for u in range(8):
            i = i8 * 8 + u
            rows_v[i, pl.ds(0, EDGE_DIM)] = attr0[i, :]
        return carry2

    lax.fori_loop(0, TAIL // 8, widen_t, 0)
    pltpu.sync_copy(rows_v.at[pl.ds(0, TAIL)], s_acc.at[idxt], add=True)

    plsc.subcore_barrier()
    pltpu.sync_copy(s_acc.at[pl.ds(row0, STRIPE)], s_out.at[c, pl.ds(row0, STRIPE)])


@functools.partial(
    pl.kernel,
    mesh=_sc_mesh(),
    out_type=jax.ShapeDtypeStruct((NC, N_ACC, HIDDEN), jnp.float32),
    scratch_types=[
        pltpu.VMEM((CH,), jnp.int32),
        pltpu.VMEM((CH,), jnp.int32),
        pltpu.VMEM((TAIL,), jnp.int32),
        pltpu.VMEM((BLK, HIDDEN), jnp.float32),
        pltpu.VMEM((BLK, HIDDEN), jnp.float32),
        pltpu.VMEM_SHARED((N_ACC, HIDDEN), jnp.float32),
        pltpu.SemaphoreType.DMA,
        pltpu.SemaphoreType.DMA,
        pltpu.SemaphoreType.DMA,
        pltpu.SemaphoreType.DMA,
    ],
)
def _sc_scatter_msg(dst_hbm, msg_hbm, zs_hbm, n_out,
                    idx0, idx1, idxt, rows0, rows1, acc,
                    si0, si1, sr0, sr1):
    c = lax.axis_index("c")
    s = lax.axis_index("s")
    row0 = s * STRIPE
    pltpu.sync_copy(zs_hbm, acc.at[pl.ds(row0, STRIPE)])

    base = (c * NS + s) * EPC

    def start(blk, idx_v, rows_v, sem_i, sem_r):
        off = pl.multiple_of(base + blk * BLK, 8)
        pltpu.make_async_copy(dst_hbm.at[pl.ds(off, CH)], idx_v, sem_i).start()
        pltpu.make_async_copy(msg_hbm.at[pl.ds(off, BLK)], rows_v, sem_r).start()

    def wait(idx_v, rows_v, sem_i, sem_r):
        pltpu.make_async_copy(dst_hbm.at[pl.ds(0, CH)], idx_v, sem_i).wait()
        pltpu.make_async_copy(msg_hbm.at[pl.ds(0, BLK)], rows_v, sem_r).wait()

    def scatter(idx_v, rows_v):
        pltpu.sync_copy(rows_v, acc.at[idx_v], add=True)

    start(0, idx0, rows0, si0, sr0)
    start(1, idx1, rows1, si1, sr1)
    plsc.subcore_barrier()

    def body(kk, carry):
        j0 = 2 * kk
        wait(idx0, rows0, si0, sr0)
        scatter(idx0, rows0)

        @pl.when(j0 + 2 < NB)
        def _():
            start(j0 + 2, idx0, rows0, si0, sr0)

        wait(idx1, rows1, si1, sr1)
        scatter(idx1, rows1)

        @pl.when(j0 + 3 < NB)
        def _():
            start(j0 + 3, idx1, rows1, si1, sr1)

        return carry

    lax.fori_loop(0, NB // 2, body, 0)

    toff = pl.multiple_of(base + NB * BLK, 8)
    pltpu.sync_copy(dst_hbm.at[pl.ds(toff, TAIL)], idxt)
    pltpu.sync_copy(msg_hbm.at[pl.ds(toff, TAIL)], rows0.at[pl.ds(0, TAIL)])
    pltpu.sync_copy(rows0.at[pl.ds(0, TAIL)], acc.at[idxt], add=True)

    plsc.subcore_barrier()
    pltpu.sync_copy(acc.at[pl.ds(row0, STRIPE)], n_out.at[c, pl.ds(row0, STRIPE)])


def _mlp(x, w1, b1, w2, b2, w3, b3):
    bf = jnp.bfloat16
    h = jnp.maximum(jnp.dot(x, w1, preferred_element_type=jnp.float32) + b1, 0.0)
    h = jnp.maximum(
        jnp.dot(h.astype(bf), w2.astype(bf), preferred_element_type=jnp.float32)
        + b2, 0.0)
    return (jnp.dot(h.astype(bf), w3.astype(bf), preferred_element_type=jnp.float32)
            + b3)


def _edge_mlp_body(ea_ref, w1_ref, b1_ref, w2_ref, b2_ref, w3_ref, b3_ref, out_ref):
    out_ref[...] = _mlp(ea_ref[...], w1_ref[...], b1_ref[...], w2_ref[...],
                        b2_ref[...], w3_ref[...], b3_ref[...])


def _loop_mlp_body(st_ref, w1_ref, b1_ref, w2_ref, b2_ref, w3_ref, b3_ref,
                   out_ref):
    st = st_ref[0] + st_ref[1]
    ssum = st[:, 0:EDGE_DIM]
    deg = st[:, EDGE_DIM:EDGE_DIM + 1]
    ea = ssum / jnp.maximum(deg, 1.0)
    out_ref[...] = _mlp(ea, w1_ref[...], b1_ref[...], w2_ref[...],
                        b2_ref[...], w3_ref[...], b3_ref[...])


def _pool_head_body(p_ref, sm_ref, batch_ref, wc1_ref, bc1_ref, wc2_ref, bc2_ref,
                    out_ref, g_ref):
    i = pl.program_id(0)

    @pl.when(i == 0)
    def _():
        g_ref[...] = jnp.full((NUM_GRAPHS, HIDDEN), -jnp.inf, jnp.float32)

    x = p_ref[0] + p_ref[1] + sm_ref[...]
    b_col = batch_ref[...]
    bmin = b_col[0, 0]
    bmax = b_col[POOL_TILE - 1, 0]
    neg_inf = jnp.float32(-jnp.inf)
    for g in range(NUM_GRAPHS):
        @pl.when((bmin <= g) & (g <= bmax))
        def _(g=g):
            pen = jnp.where(b_col == g, 0.0, neg_inf)
            gmax = jnp.max(x + pen, axis=0, keepdims=True)
            g_ref[pl.ds(g, 1), :] = jnp.maximum(g_ref[pl.ds(g, 1), :], gmax)

    @pl.when(i == N_ACC // POOL_TILE - 1)
    def _():
        g = g_ref[...]
        gh = jnp.maximum(
            jnp.dot(g, wc1_ref[...], preferred_element_type=jnp.float32)
            + bc1_ref[...], 0.0)
        out_ref[...] = (jnp.dot(gh, wc2_ref[...], preferred_element_type=jnp.float32)
                        + bc2_ref[...])


def kernel(edge_index, edge_attr, batch, W1, b1, W2, b2, W3, b3, Wc1, bc1, Wc2, bc2):
    dst = edge_index[1].astype(jnp.int32)
    batch_pad = jnp.concatenate(
        [batch.astype(jnp.int32),
         jnp.full((N_ACC - N_NODES,), NUM_GRAPHS, jnp.int32)])
    batch_col = batch_pad.reshape(N_ACC, 1)

    init_rows = jnp.zeros((BLK, HIDDEN), jnp.float32).at[:, EDGE_DIM].set(1.0)
    zstripe = jnp.zeros((STRIPE, HIDDEN), jnp.float32)

    b1r = b1.reshape(1, HIDDEN)
    b2r = b2.reshape(1, HIDDEN)
    b3r = b3.reshape(1, HIDDEN)
    bc1r = bc1.reshape(1, HIDDEN)
    bc2r = bc2.reshape(1, NUM_CLASSES)

    stats = _sc_edge_stats(dst, edge_attr, init_rows, zstripe)

    msg = pl.pallas_call(
        _edge_mlp_body,
        grid=(N_EDGES // MLP_TILE,),
        in_specs=[
            pl.BlockSpec((MLP_TILE, EDGE_DIM), lambda i: (i, 0)),
            pl.BlockSpec((EDGE_DIM, HIDDEN), lambda i: (0, 0)),
            pl.BlockSpec((1, HIDDEN), lambda i: (0, 0)),
            pl.BlockSpec((HIDDEN, HIDDEN), lambda i: (0, 0)),
            pl.BlockSpec((1, HIDDEN), lambda i: (0, 0)),
            pl.BlockSpec((HIDDEN, HIDDEN), lambda i: (0, 0)),
            pl.BlockSpec((1, HIDDEN), lambda i: (0, 0)),
        ],
        out_specs=pl.BlockSpec((MLP_TILE, HIDDEN), lambda i: (i, 0)),
        out_shape=jax.ShapeDtypeStruct((N_EDGES, HIDDEN), jnp.float32),
    )(edge_attr, W1, b1r, W2, b2r, W3, b3r)

    self_msg = pl.pallas_call(
        _loop_mlp_body,
        grid=(N_ACC // LOOP_TILE,),
        in_specs=[
            pl.BlockSpec((NC, LOOP_TILE, HIDDEN), lambda i: (0, i, 0)),
            pl.BlockSpec((EDGE_DIM, HIDDEN), lambda i: (0, 0)),
            pl.BlockSpec((1, HIDDEN), lambda i: (0, 0)),
            pl.BlockSpec((HIDDEN, HIDDEN), lambda i: (0, 0)),
            pl.BlockSpec((1, HIDDEN), lambda i: (0, 0)),
            pl.BlockSpec((HIDDEN, HIDDEN), lambda i: (0, 0)),
            pl.BlockSpec((1, HIDDEN), lambda i: (0, 0)),
        ],
        out_specs=pl.BlockSpec((LOOP_TILE, HIDDEN), lambda i: (i, 0)),
        out_shape=jax.ShapeDtypeStruct((N_ACC, HIDDEN), jnp.float32),
    )(stats, W1, b1r, W2, b2r, W3, b3r)

    n_parts = _sc_scatter_msg(dst, msg, zstripe)

    logits = pl.pallas_call(
        _pool_head_body,
        grid=(N_ACC // POOL_TILE,),
        in_specs=[
            pl.BlockSpec((NC, POOL_TILE, HIDDEN), lambda i: (0, i, 0)),
            pl.BlockSpec((POOL_TILE, HIDDEN), lambda i: (i, 0)),
            pl.BlockSpec((POOL_TILE, 1), lambda i: (i, 0)),
            pl.BlockSpec((HIDDEN, HIDDEN), lambda i: (0, 0)),
            pl.BlockSpec((1, HIDDEN), lambda i: (0, 0)),
            pl.BlockSpec((HIDDEN, NUM_CLASSES), lambda i: (0, 0)),
            pl.BlockSpec((1, NUM_CLASSES), lambda i: (0, 0)),
        ],
        out_specs=pl.BlockSpec((NUM_GRAPHS, NUM_CLASSES), lambda i: (0, 0)),
        out_shape=jax.ShapeDtypeStruct((NUM_GRAPHS, NUM_CLASSES), jnp.float32),
        scratch_shapes=[pltpu.VMEM((NUM_GRAPHS, HIDDEN), jnp.float32)],
    )(n_parts, self_msg, batch_col, Wc1, bc1r, Wc2, bc2r)

    return logits


# speedup vs baseline: 1.0370x; 1.0370x over previous
import functools

import jax
import jax.numpy as jnp
from jax import lax
from jax.experimental import pallas as pl
from jax.experimental.pallas import tpu as pltpu
from jax.experimental.pallas import tpu_sc as plsc

N_NODES = 10000
N_EDGES = 640000
EDGE_DIM = 16
HIDDEN = 128
NUM_CLASSES = 10
NUM_GRAPHS = 64

NC = 2
NS = 16
NW = NC * NS

CH = 128
BLK = CH
E_HALF = N_EDGES // 2
EPC = E_HALF // NW
NB = EPC // BLK
TAIL = EPC - NB * BLK
N_ACC = 10112
STRIPE = N_ACC // NS

MLP_TILE = 2000
LOOP_TILE = 1264
POOL_TILE = 1264


def _sc_mesh():
    return plsc.VectorSubcoreMesh(core_axis_name="c", subcore_axis_name="s")


@functools.partial(
    pl.kernel,
    mesh=_sc_mesh(),
    out_type=jax.ShapeDtypeStruct((NC, N_ACC, HIDDEN), jnp.float32),
    scratch_types=[
        pltpu.VMEM((CH,), jnp.int32),
        pltpu.VMEM((CH,), jnp.int32),
        pltpu.VMEM((TAIL,), jnp.int32),
        pltpu.VMEM((BLK, EDGE_DIM), jnp.float32),
        pltpu.VMEM((BLK, EDGE_DIM), jnp.float32),
        pltpu.VMEM((BLK, HIDDEN), jnp.float32),
        pltpu.VMEM_SHARED((N_ACC, HIDDEN), jnp.float32),
        pltpu.SemaphoreType.DMA,
        pltpu.SemaphoreType.DMA,
        pltpu.SemaphoreType.DMA,
        pltpu.SemaphoreType.DMA,
    ],
)
def _sc_edge_stats(dst_hbm, ea_hbm, init_hbm, zs_hbm, s_out,
                   idx0, idx1, idxt, attr0, attr1, rows_v, s_acc,
                   si0, si1, sa0, sa1):
    c = lax.axis_index("c")
    s = lax.axis_index("s")
    row0 = s * STRIPE
    pltpu.sync_copy(zs_hbm, s_acc.at[pl.ds(row0, STRIPE)])
    pltpu.sync_copy(init_hbm, rows_v)

    base = (c * NS + s) * EPC

    def start(blk, idx_v, attr_v, sem_i, sem_a):
        off = pl.multiple_of(base + blk * BLK, 8)
        pltpu.make_async_copy(dst_hbm.at[pl.ds(off, CH)], idx_v, sem_i).start()
        pltpu.make_async_copy(ea_hbm.at[pl.ds(off, BLK)], attr_v, sem_a).start()

    def wait(idx_v, attr_v, sem_i, sem_a):
        pltpu.make_async_copy(dst_hbm.at[pl.ds(0, CH)], idx_v, sem_i).wait()
        pltpu.make_async_copy(ea_hbm.at[pl.ds(0, BLK)], attr_v, sem_a).wait()

    def process(idx_v, attr_v):
        def widen(i8, carry2):
            for u in range(8):
                i = i8 * 8 + u
                rows_v[i, pl.ds(0, EDGE_DIM)] = attr_v[i, :]
            return carry2

        lax.fori_loop(0, BLK // 8, widen, 0)
        pltpu.sync_copy(rows_v, s_acc.at[idx_v], add=True)

    start(0, idx0, attr0, si0, sa0)
    start(1, idx1, attr1, si1, sa1)
    plsc.subcore_barrier()

    def body(kk, carry):
        j0 = 2 * kk
        wait(idx0, attr0, si0, sa0)
        process(idx0, attr0)

        @pl.when(j0 + 2 < NB)
        def _():
            start(j0 + 2, idx0, attr0, si0, sa0)

        wait(idx1, attr1, si1, sa1)
        process(idx1, attr1)

        @pl.when(j0 + 3 < NB)
        def _():
            start(j0 + 3, idx1, attr1, si1, sa1)

        return carry

    lax.fori_loop(0, NB // 2, body, 0)

    toff = pl.multiple_of(base + NB * BLK, 8)
    pltpu.sync_copy(dst_hbm.at[pl.ds(toff, TAIL)], idxt)
    pltpu.sync_copy(ea_hbm.at[pl.ds(toff, TAIL)], attr0.at[pl.ds(0, TAIL)])

    def widen_t(i8, carry2):
        for u in range(8):
            i = i8 * 8 + u
            rows_v[i, pl.ds(0, EDGE_DIM)] = attr0[i, :]
        return carry2

    lax.fori_loop(0, TAIL // 8, widen_t, 0)
    pltpu.sync_copy(rows_v.at[pl.ds(0, TAIL)], s_acc.at[idxt], add=True)

    plsc.subcore_barrier()
    pltpu.sync_copy(s_acc.at[pl.ds(row0, STRIPE)], s_out.at[c, pl.ds(row0, STRIPE)])


@functools.partial(
    pl.kernel,
    mesh=_sc_mesh(),
    out_type=jax.ShapeDtypeStruct((NC, N_ACC, HIDDEN), jnp.float32),
    scratch_types=[
        pltpu.VMEM((CH,), jnp.int32),
        pltpu.VMEM((CH,), jnp.int32),
        pltpu.VMEM((TAIL,), jnp.int32),
        pltpu.VMEM((BLK, HIDDEN), jnp.float32),
        pltpu.VMEM((BLK, HIDDEN), jnp.float32),
        pltpu.VMEM_SHARED((N_ACC, HIDDEN), jnp.float32),
        pltpu.SemaphoreType.DMA,
        pltpu.SemaphoreType.DMA,
        pltpu.SemaphoreType.DMA,
        pltpu.SemaphoreType.DMA,
    ],
)
def _sc_scatter_msg(dst_hbm, msg_hbm, zs_hbm, n_out,
                    idx0, idx1, idxt, rows0, rows1, acc,
                    si0, si1, sr0, sr1):
    c = lax.axis_index("c")
    s = lax.axis_index("s")
    row0 = s * STRIPE
    pltpu.sync_copy(zs_hbm, acc.at[pl.ds(row0, STRIPE)])

    base = (c * NS + s) * EPC

    def start(blk, idx_v, rows_v, sem_i, sem_r):
        off = pl.multiple_of(base + blk * BLK, 8)
        pltpu.make_async_copy(dst_hbm.at[pl.ds(off, CH)], idx_v, sem_i).start()
        pltpu.make_async_copy(msg_hbm.at[pl.ds(off, BLK)], rows_v, sem_r).start()

    def wait(idx_v, rows_v, sem_i, sem_r):
        pltpu.make_async_copy(dst_hbm.at[pl.ds(0, CH)], idx_v, sem_i).wait()
        pltpu.make_async_copy(msg_hbm.at[pl.ds(0, BLK)], rows_v, sem_r).wait()

    def scatter(idx_v, rows_v):
        pltpu.sync_copy(rows_v, acc.at[idx_v], add=True)

    start(0, idx0, rows0, si0, sr0)
    start(1, idx1, rows1, si1, sr1)
    plsc.subcore_barrier()

    def body(kk, carry):
        j0 = 2 * kk
        wait(idx0, rows0, si0, sr0)
        scatter(idx0, rows0)

        @pl.when(j0 + 2 < NB)
        def _():
            start(j0 + 2, idx0, rows0, si0, sr0)

        wait(idx1, rows1, si1, sr1)
        scatter(idx1, rows1)

        @pl.when(j0 + 3 < NB)
        def _():
            start(j0 + 3, idx1, rows1, si1, sr1)

        return carry

    lax.fori_loop(0, NB // 2, body, 0)

    toff = pl.multiple_of(base + NB * BLK, 8)
    pltpu.sync_copy(dst_hbm.at[pl.ds(toff, TAIL)], idxt)
    pltpu.sync_copy(msg_hbm.at[pl.ds(toff, TAIL)], rows0.at[pl.ds(0, TAIL)])
    pltpu.sync_copy(rows0.at[pl.ds(0, TAIL)], acc.at[idxt], add=True)

    plsc.subcore_barrier()
    pltpu.sync_copy(acc.at[pl.ds(row0, STRIPE)], n_out.at[c, pl.ds(row0, STRIPE)])


def _mlp(x, w1, b1, w2, b2, w3, b3):
    h = jnp.maximum(jnp.dot(x, w1, preferred_element_type=jnp.float32) + b1, 0.0)
    h = jnp.maximum(jnp.dot(h, w2, preferred_element_type=jnp.float32) + b2, 0.0)
    return jnp.dot(h, w3, preferred_element_type=jnp.float32) + b3


def _edge_mlp_body(ea_ref, w1_ref, b1_ref, w2_ref, b2_ref, w3_ref, b3_ref, out_ref):
    out_ref[...] = _mlp(ea_ref[...], w1_ref[...], b1_ref[...], w2_ref[...],
                        b2_ref[...], w3_ref[...], b3_ref[...])


def _loop_mlp_body(st_ref, su_ref, w1_ref, b1_ref, w2_ref, b2_ref, w3_ref, b3_ref,
                   out_ref):
    st = (st_ref[0] + st_ref[1]) + (su_ref[0] + su_ref[1])
    ssum = st[:, 0:EDGE_DIM]
    deg = st[:, EDGE_DIM:EDGE_DIM + 1]
    ea = ssum / jnp.maximum(deg, 1.0)
    out_ref[...] = _mlp(ea, w1_ref[...], b1_ref[...], w2_ref[...],
                        b2_ref[...], w3_ref[...], b3_ref[...])


def _pool_head_body(p_ref, q_ref, sm_ref, batch_ref, wc1_ref, bc1_ref, wc2_ref,
                    bc2_ref, out_ref, g_ref):
    i = pl.program_id(0)

    @pl.when(i == 0)
    def _():
        g_ref[...] = jnp.full((NUM_GRAPHS, HIDDEN), -jnp.inf, jnp.float32)

    x = (p_ref[0] + p_ref[1]) + (q_ref[0] + q_ref[1]) + sm_ref[...]
    b_col = batch_ref[...]
    bmin = b_col[0, 0]
    bmax = b_col[POOL_TILE - 1, 0]
    neg_inf = jnp.float32(-jnp.inf)
    for g in range(NUM_GRAPHS):
        @pl.when((bmin <= g) & (g <= bmax))
        def _(g=g):
            pen = jnp.where(b_col == g, 0.0, neg_inf)
            gmax = jnp.max(x + pen, axis=0, keepdims=True)
            g_ref[pl.ds(g, 1), :] = jnp.maximum(g_ref[pl.ds(g, 1), :], gmax)

    @pl.when(i == N_ACC // POOL_TILE - 1)
    def _():
        g = g_ref[...]
        gh = jnp.maximum(
            jnp.dot(g, wc1_ref[...], preferred_element_type=jnp.float32)
            + bc1_ref[...], 0.0)
        out_ref[...] = (jnp.dot(gh, wc2_ref[...], preferred_element_type=jnp.float32)
                        + bc2_ref[...])


def kernel(edge_index, edge_attr, batch, W1, b1, W2, b2, W3, b3, Wc1, bc1, Wc2, bc2):
    dst = edge_index[1].astype(jnp.int32)
    batch_pad = jnp.concatenate(
        [batch.astype(jnp.int32),
         jnp.full((N_ACC - N_NODES,), NUM_GRAPHS, jnp.int32)])
    batch_col = batch_pad.reshape(N_ACC, 1)

    init_rows = jnp.zeros((BLK, HIDDEN), jnp.float32).at[:, EDGE_DIM].set(1.0)
    zstripe = jnp.zeros((STRIPE, HIDDEN), jnp.float32)

    b1r = b1.reshape(1, HIDDEN)
    b2r = b2.reshape(1, HIDDEN)
    b3r = b3.reshape(1, HIDDEN)
    bc1r = bc1.reshape(1, HIDDEN)
    bc2r = bc2.reshape(1, NUM_CLASSES)

    stats_a = _sc_edge_stats(dst[:E_HALF], edge_attr[:E_HALF], init_rows, zstripe)
    stats_b = _sc_edge_stats(dst[E_HALF:], edge_attr[E_HALF:], init_rows, zstripe)

    def _edge_mlp(ea):
        return pl.pallas_call(
            _edge_mlp_body,
            grid=(E_HALF // MLP_TILE,),
            in_specs=[
                pl.BlockSpec((MLP_TILE, EDGE_DIM), lambda i: (i, 0)),
                pl.BlockSpec((EDGE_DIM, HIDDEN), lambda i: (0, 0)),
                pl.BlockSpec((1, HIDDEN), lambda i: (0, 0)),
                pl.BlockSpec((HIDDEN, HIDDEN), lambda i: (0, 0)),
                pl.BlockSpec((1, HIDDEN), lambda i: (0, 0)),
                pl.BlockSpec((HIDDEN, HIDDEN), lambda i: (0, 0)),
                pl.BlockSpec((1, HIDDEN), lambda i: (0, 0)),
            ],
            out_specs=pl.BlockSpec((MLP_TILE, HIDDEN), lambda i: (i, 0)),
            out_shape=jax.ShapeDtypeStruct((E_HALF, HIDDEN), jnp.float32),
        )(ea, W1, b1r, W2, b2r, W3, b3r)

    msg_a = _edge_mlp(edge_attr[:E_HALF])
    msg_b = _edge_mlp(edge_attr[E_HALF:])

    self_msg = pl.pallas_call(
        _loop_mlp_body,
        grid=(N_ACC // LOOP_TILE,),
        in_specs=[
            pl.BlockSpec((NC, LOOP_TILE, HIDDEN), lambda i: (0, i, 0)),
            pl.BlockSpec((NC, LOOP_TILE, HIDDEN), lambda i: (0, i, 0)),
            pl.BlockSpec((EDGE_DIM, HIDDEN), lambda i: (0, 0)),
            pl.BlockSpec((1, HIDDEN), lambda i: (0, 0)),
            pl.BlockSpec((HIDDEN, HIDDEN), lambda i: (0, 0)),
            pl.BlockSpec((1, HIDDEN), lambda i: (0, 0)),
            pl.BlockSpec((HIDDEN, HIDDEN), lambda i: (0, 0)),
            pl.BlockSpec((1, HIDDEN), lambda i: (0, 0)),
        ],
        out_specs=pl.BlockSpec((LOOP_TILE, HIDDEN), lambda i: (i, 0)),
        out_shape=jax.ShapeDtypeStruct((N_ACC, HIDDEN), jnp.float32),
    )(stats_a, stats_b, W1, b1r, W2, b2r, W3, b3r)

    dst_a = dst[:E_HALF]
    dst_b = dst[E_HALF:]
    n_parts_a = _sc_scatter_msg(dst_a, msg_a, zstripe)
    n_parts_b = _sc_scatter_msg(dst_b, msg_b, zstripe)

    logits = pl.pallas_call(
        _pool_head_body,
        grid=(N_ACC // POOL_TILE,),
        in_specs=[
            pl.BlockSpec((NC, POOL_TILE, HIDDEN), lambda i: (0, i, 0)),
            pl.BlockSpec((NC, POOL_TILE, HIDDEN), lambda i: (0, i, 0)),
            pl.BlockSpec((POOL_TILE, HIDDEN), lambda i: (i, 0)),
            pl.BlockSpec((POOL_TILE, 1), lambda i: (i, 0)),
            pl.BlockSpec((HIDDEN, HIDDEN), lambda i: (0, 0)),
            pl.BlockSpec((1, HIDDEN), lambda i: (0, 0)),
            pl.BlockSpec((HIDDEN, NUM_CLASSES), lambda i: (0, 0)),
            pl.BlockSpec((1, NUM_CLASSES), lambda i: (0, 0)),
        ],
        out_specs=pl.BlockSpec((NUM_GRAPHS, NUM_CLASSES), lambda i: (0, 0)),
        out_shape=jax.ShapeDtypeStruct((NUM_GRAPHS, NUM_CLASSES), jnp.float32),
        scratch_shapes=[pltpu.VMEM((NUM_GRAPHS, HIDDEN), jnp.float32)],
    )(n_parts_a, n_parts_b, self_msg, batch_col, Wc1, bc1r, Wc2, bc2r)

    return logits


# speedup vs baseline: 1.0387x; 1.0016x over previous
import functools

import jax
import jax.numpy as jnp
from jax import lax
from jax.experimental import pallas as pl
from jax.experimental.pallas import tpu as pltpu
from jax.experimental.pallas import tpu_sc as plsc

N_NODES = 10000
N_EDGES = 640000
EDGE_DIM = 16
HIDDEN = 128
NUM_CLASSES = 10
NUM_GRAPHS = 64

NC = 2
NS = 16
NW = NC * NS

CH = 128
BLK = CH
EPC = N_EDGES // NW
NB = EPC // BLK
TAIL = EPC - NB * BLK
N_ACC = 10112
STRIPE = N_ACC // NS

MLP_TILE = 2000
LOOP_TILE = 1264
POOL_TILE = 1264


def _sc_mesh():
    return plsc.VectorSubcoreMesh(core_axis_name="c", subcore_axis_name="s")


@functools.partial(
    pl.kernel,
    mesh=_sc_mesh(),
    out_type=jax.ShapeDtypeStruct((NC, N_ACC, HIDDEN), jnp.float32),
    scratch_types=[
        pltpu.VMEM((CH,), jnp.int32),
        pltpu.VMEM((CH,), jnp.int32),
        pltpu.VMEM((TAIL,), jnp.int32),
        pltpu.VMEM((BLK, EDGE_DIM), jnp.float32),
        pltpu.VMEM((BLK, EDGE_DIM), jnp.float32),
        pltpu.VMEM((BLK, HIDDEN), jnp.float32),
        pltpu.VMEM_SHARED((N_ACC, HIDDEN), jnp.float32),
        pltpu.SemaphoreType.DMA,
        pltpu.SemaphoreType.DMA,
        pltpu.SemaphoreType.DMA,
        pltpu.SemaphoreType.DMA,
    ],
)
def _sc_edge_stats(dst_hbm, ea_hbm, init_hbm, zs_hbm, s_out,
                   idx0, idx1, idxt, attr0, attr1, rows_v, s_acc,
                   si0, si1, sa0, sa1):
    c = lax.axis_index("c")
    s = lax.axis_index("s")
    row0 = s * STRIPE
    pltpu.sync_copy(zs_hbm, s_acc.at[pl.ds(row0, STRIPE)])
    pltpu.sync_copy(init_hbm, rows_v)

    base = (c * NS + s) * EPC

    def start(blk, idx_v, attr_v, sem_i, sem_a):
        off = pl.multiple_of(base + blk * BLK, 8)
        pltpu.make_async_copy(dst_hbm.at[pl.ds(off, CH)], idx_v, sem_i).start()
        pltpu.make_async_copy(ea_hbm.at[pl.ds(off, BLK)], attr_v, sem_a).start()

    def wait(idx_v, attr_v, sem_i, sem_a):
        pltpu.make_async_copy(dst_hbm.at[pl.ds(0, CH)], idx_v, sem_i).wait()
        pltpu.make_async_copy(ea_hbm.at[pl.ds(0, BLK)], attr_v, sem_a).wait()

    def process(idx_v, attr_v):
        def widen(i8, carry2):
            for u in range(8):
                i = i8 * 8 + u
                rows_v[i, pl.ds(0, EDGE_DIM)] = attr_v[i, :]
            return carry2

        lax.fori_loop(0, BLK // 8, widen, 0)
        pltpu.sync_copy(rows_v, s_acc.at[idx_v], add=True)

    start(0, idx0, attr0, si0, sa0)
    start(1, idx1, attr1, si1, sa1)
    plsc.subcore_barrier()

    def body(kk, carry):
        j0 = 2 * kk
        wait(idx0, attr0, si0, sa0)
        process(idx0, attr0)

        @pl.when(j0 + 2 < NB)
        def _():
            start(j0 + 2, idx0, attr0, si0, sa0)

        wait(idx1, attr1, si1, sa1)
        process(idx1, attr1)

        @pl.when(j0 + 3 < NB)
        def _():
            start(j0 + 3, idx1, attr1, si1, sa1)

        return carry

    lax.fori_loop(0, NB // 2, body, 0)

    toff = pl.multiple_of(base + NB * BLK, 8)
    pltpu.sync_copy(dst_hbm.at[pl.ds(toff, TAIL)], idxt)
    pltpu.sync_copy(ea_hbm.at[pl.ds(toff, TAIL)], attr0.at[pl.ds(0, TAIL)])

    def widen_t(i8, carry2):
        for u in range(8):
            i = i8 * 8 + u
            rows_v[i, pl.ds(0, EDGE_DIM)] = attr0[i, :]
        return carry2

    lax.fori_loop(0, TAIL // 8, widen_t, 0)
    pltpu.sync_copy(rows_v.at[pl.ds(0, TAIL)], s_acc.at[idxt], add=True)

    plsc.subcore_barrier()
    pltpu.sync_copy(s_acc.at[pl.ds(row0, STRIPE)], s_out.at[c, pl.ds(row0, STRIPE)])


@functools.partial(
    pl.kernel,
    mesh=_sc_mesh(),
    out_type=jax.ShapeDtypeStruct((NC, N_ACC, HIDDEN), jnp.float32),
    scratch_types=[
        pltpu.VMEM((CH,), jnp.int32),
        pltpu.VMEM((CH,), jnp.int32),
        pltpu.VMEM((TAIL,), jnp.int32),
        pltpu.VMEM((BLK, HIDDEN), jnp.float32),
        pltpu.VMEM((BLK, HIDDEN), jnp.float32),
        pltpu.VMEM_SHARED((N_ACC, HIDDEN), jnp.float32),
        pltpu.SemaphoreType.DMA,
        pltpu.SemaphoreType.DMA,
        pltpu.SemaphoreType.DMA,
        pltpu.SemaphoreType.DMA,
    ],
)
def _sc_scatter_msg(dst_hbm, msg_hbm, zs_hbm, n_out,
                    idx0, idx1, idxt, rows0, rows1, acc,
                    si0, si1, sr0, sr1):
    c = lax.axis_index("c")
    s = lax.axis_index("s")
    row0 = s * STRIPE
    pltpu.sync_copy(zs_hbm, acc.at[pl.ds(row0, STRIPE)])

    base = (c * NS + s) * EPC

    def start(blk, idx_v, rows_v, sem_i, sem_r):
        off = pl.multiple_of(base + blk * BLK, 8)
        pltpu.make_async_copy(dst_hbm.at[pl.ds(off, CH)], idx_v, sem_i).start()
        pltpu.make_async_copy(msg_hbm.at[pl.ds(off, BLK)], rows_v, sem_r).start()

    def wait(idx_v, rows_v, sem_i, sem_r):
        pltpu.make_async_copy(dst_hbm.at[pl.ds(0, CH)], idx_v, sem_i).wait()
        pltpu.make_async_copy(msg_hbm.at[pl.ds(0, BLK)], rows_v, sem_r).wait()

    def scatter(idx_v, rows_v):
        pltpu.sync_copy(rows_v, acc.at[idx_v], add=True)

    start(0, idx0, rows0, si0, sr0)
    start(1, idx1, rows1, si1, sr1)
    plsc.subcore_barrier()

    def body(kk, carry):
        j0 = 2 * kk
        wait(idx0, rows0, si0, sr0)
        scatter(idx0, rows0)

        @pl.when(j0 + 2 < NB)
        def _():
            start(j0 + 2, idx0, rows0, si0, sr0)

        wait(idx1, rows1, si1, sr1)
        scatter(idx1, rows1)

        @pl.when(j0 + 3 < NB)
        def _():
            start(j0 + 3, idx1, rows1, si1, sr1)

        return carry

    lax.fori_loop(0, NB // 2, body, 0)

    toff = pl.multiple_of(base + NB * BLK, 8)
    pltpu.sync_copy(dst_hbm.at[pl.ds(toff, TAIL)], idxt)
    pltpu.sync_copy(msg_hbm.at[pl.ds(toff, TAIL)], rows0.at[pl.ds(0, TAIL)])
    pltpu.sync_copy(rows0.at[pl.ds(0, TAIL)], acc.at[idxt], add=True)

    plsc.subcore_barrier()
    pltpu.sync_copy(acc.at[pl.ds(row0, STRIPE)], n_out.at[c, pl.ds(row0, STRIPE)])


def _mlp(x, w1, b1, w2, b2, w3, b3):
    h = jnp.maximum(jnp.dot(x, w1, preferred_element_type=jnp.float32) + b1, 0.0)
    h = jnp.maximum(jnp.dot(h, w2, preferred_element_type=jnp.float32) + b2, 0.0)
    return jnp.dot(h, w3, preferred_element_type=jnp.float32) + b3


def _edge_mlp_body(ea_ref, w1_ref, b1_ref, w2_ref, b2_ref, w3_ref, b3_ref, out_ref):
    out_ref[...] = _mlp(ea_ref[...], w1_ref[...], b1_ref[...], w2_ref[...],
                        b2_ref[...], w3_ref[...], b3_ref[...])


def _loop_mlp_body(st_ref, w1_ref, b1_ref, w2_ref, b2_ref, w3_ref, b3_ref,
                   out_ref):
    st = st_ref[0] + st_ref[1]
    ssum = st[:, 0:EDGE_DIM]
    deg = st[:, EDGE_DIM:EDGE_DIM + 1]
    ea = ssum / jnp.maximum(deg, 1.0)
    out_ref[...] = _mlp(ea, w1_ref[...], b1_ref[...], w2_ref[...],
                        b2_ref[...], w3_ref[...], b3_ref[...])


def _pool_head_body(p_ref, sm_ref, batch_ref, wc1_ref, bc1_ref, wc2_ref, bc2_ref,
                    out_ref, g_ref):
    i = pl.program_id(0)

    @pl.when(i == 0)
    def _():
        g_ref[...] = jnp.full((NUM_GRAPHS, HIDDEN), -jnp.inf, jnp.float32)

    x = p_ref[0] + p_ref[1] + sm_ref[...]
    b_col = batch_ref[...]
    bmin = b_col[0, 0]
    bmax = b_col[POOL_TILE - 1, 0]
    neg_inf = jnp.float32(-jnp.inf)
    for g in range(NUM_GRAPHS):
        @pl.when((bmin <= g) & (g <= bmax))
        def _(g=g):
            pen = jnp.where(b_col == g, 0.0, neg_inf)
            gmax = jnp.max(x + pen, axis=0, keepdims=True)
            g_ref[pl.ds(g, 1), :] = jnp.maximum(g_ref[pl.ds(g, 1), :], gmax)

    @pl.when(i == N_ACC // POOL_TILE - 1)
    def _():
        g = g_ref[...]
        gh = jnp.maximum(
            jnp.dot(g, wc1_ref[...], preferred_element_type=jnp.float32)
            + bc1_ref[...], 0.0)
        out_ref[...] = (jnp.dot(gh, wc2_ref[...], preferred_element_type=jnp.float32)
                        + bc2_ref[...])


def kernel(edge_index, edge_attr, batch, W1, b1, W2, b2, W3, b3, Wc1, bc1, Wc2, bc2):
    dst = edge_index[1].astype(jnp.int32)
    batch_pad = jnp.concatenate(
        [batch.astype(jnp.int32),
         jnp.full((N_ACC - N_NODES,), NUM_GRAPHS, jnp.int32)])
    batch_col = batch_pad.reshape(N_ACC, 1)

    init_rows = jnp.zeros((BLK, HIDDEN), jnp.float32).at[:, EDGE_DIM].set(1.0)
    zstripe = jnp.zeros((STRIPE, HIDDEN), jnp.float32)

    b1r = b1.reshape(1, HIDDEN)
    b2r = b2.reshape(1, HIDDEN)
    b3r = b3.reshape(1, HIDDEN)
    bc1r = bc1.reshape(1, HIDDEN)
    bc2r = bc2.reshape(1, NUM_CLASSES)

    stats = _sc_edge_stats(dst, edge_attr, init_rows, zstripe)

    msg = pl.pallas_call(
        _edge_mlp_body,
        grid=(N_EDGES // MLP_TILE,),
        in_specs=[
            pl.BlockSpec((MLP_TILE, EDGE_DIM), lambda i: (i, 0)),
            pl.BlockSpec((EDGE_DIM, HIDDEN), lambda i: (0, 0)),
            pl.BlockSpec((1, HIDDEN), lambda i: (0, 0)),
            pl.BlockSpec((HIDDEN, HIDDEN), lambda i: (0, 0)),
            pl.BlockSpec((1, HIDDEN), lambda i: (0, 0)),
            pl.BlockSpec((HIDDEN, HIDDEN), lambda i: (0, 0)),
            pl.BlockSpec((1, HIDDEN), lambda i: (0, 0)),
        ],
        out_specs=pl.BlockSpec((MLP_TILE, HIDDEN), lambda i: (i, 0)),
        out_shape=jax.ShapeDtypeStruct((N_EDGES, HIDDEN), jnp.float32),
    )(edge_attr, W1, b1r, W2, b2r, W3, b3r)

    self_msg = pl.pallas_call(
        _loop_mlp_body,
        grid=(N_ACC // LOOP_TILE,),
        in_specs=[
            pl.BlockSpec((NC, LOOP_TILE, HIDDEN), lambda i: (0, i, 0)),
            pl.BlockSpec((EDGE_DIM, HIDDEN), lambda i: (0, 0)),
            pl.BlockSpec((1, HIDDEN), lambda i: (0, 0)),
            pl.BlockSpec((HIDDEN, HIDDEN), lambda i: (0, 0)),
            pl.BlockSpec((1, HIDDEN), lambda i: (0, 0)),
            pl.BlockSpec((HIDDEN, HIDDEN), lambda i: (0, 0)),
            pl.BlockSpec((1, HIDDEN), lambda i: (0, 0)),
        ],
        out_specs=pl.BlockSpec((LOOP_TILE, HIDDEN), lambda i: (i, 0)),
        out_shape=jax.ShapeDtypeStruct((N_ACC, HIDDEN), jnp.float32),
    )(stats, W1, b1r, W2, b2r, W3, b3r)

    n_parts = _sc_scatter_msg(dst, msg, zstripe)

    logits = pl.pallas_call(
        _pool_head_body,
        grid=(N_ACC // POOL_TILE,),
        in_specs=[
            pl.BlockSpec((NC, POOL_TILE, HIDDEN), lambda i: (0, i, 0)),
            pl.BlockSpec((POOL_TILE, HIDDEN), lambda i: (i, 0)),
            pl.BlockSpec((POOL_TILE, 1), lambda i: (i, 0)),
            pl.BlockSpec((HIDDEN, HIDDEN), lambda i: (0, 0)),
            pl.BlockSpec((1, HIDDEN), lambda i: (0, 0)),
            pl.BlockSpec((HIDDEN, NUM_CLASSES), lambda i: (0, 0)),
            pl.BlockSpec((1, NUM_CLASSES), lambda i: (0, 0)),
        ],
        out_specs=pl.BlockSpec((NUM_GRAPHS, NUM_CLASSES), lambda i: (0, 0)),
        out_shape=jax.ShapeDtypeStruct((NUM_GRAPHS, NUM_CLASSES), jnp.float32),
        scratch_shapes=[pltpu.VMEM((NUM_GRAPHS, HIDDEN), jnp.float32)],
    )(n_parts, self_msg, batch_col, Wc1, bc1r, Wc2, bc2r)

    return logits


# speedup vs baseline: 1.1203x; 1.0786x over previous
import functools

import jax
import jax.numpy as jnp
from jax import lax
from jax.experimental import pallas as pl
from jax.experimental.pallas import tpu as pltpu
from jax.experimental.pallas import tpu_sc as plsc

N_NODES = 10000
N_EDGES = 640000
EDGE_DIM = 16
HIDDEN = 128
NUM_CLASSES = 10
NUM_GRAPHS = 64

NC = 2
NS = 16
NW = NC * NS

CH = 128
BLK = CH
EPC = N_EDGES // NW
NB = EPC // BLK
TAIL = EPC - NB * BLK
N_ACC = 10112
STRIPE = N_ACC // NS

MLP_TILE = 3200
LOOP_TILE = 1264
POOL_TILE = 1264


def _sc_mesh():
    return plsc.VectorSubcoreMesh(core_axis_name="c", subcore_axis_name="s")


@functools.partial(
    pl.kernel,
    mesh=_sc_mesh(),
    out_type=jax.ShapeDtypeStruct((NC, N_ACC, HIDDEN), jnp.float32),
    scratch_types=[
        pltpu.VMEM((CH,), jnp.int32),
        pltpu.VMEM((CH,), jnp.int32),
        pltpu.VMEM((TAIL,), jnp.int32),
        pltpu.VMEM((BLK, EDGE_DIM), jnp.float32),
        pltpu.VMEM((BLK, EDGE_DIM), jnp.float32),
        pltpu.VMEM((BLK, HIDDEN), jnp.float32),
        pltpu.VMEM_SHARED((N_ACC, HIDDEN), jnp.float32),
        pltpu.SemaphoreType.DMA,
        pltpu.SemaphoreType.DMA,
        pltpu.SemaphoreType.DMA,
        pltpu.SemaphoreType.DMA,
    ],
)
def _sc_edge_stats(dst_hbm, ea_hbm, init_hbm, zs_hbm, s_out,
                   idx0, idx1, idxt, attr0, attr1, rows_v, s_acc,
                   si0, si1, sa0, sa1):
    c = lax.axis_index("c")
    s = lax.axis_index("s")
    row0 = s * STRIPE
    pltpu.sync_copy(zs_hbm, s_acc.at[pl.ds(row0, STRIPE)])
    pltpu.sync_copy(init_hbm, rows_v)

    base = (c * NS + s) * EPC

    def start(blk, idx_v, attr_v, sem_i, sem_a):
        off = pl.multiple_of(base + blk * BLK, 8)
        pltpu.make_async_copy(dst_hbm.at[pl.ds(off, CH)], idx_v, sem_i).start()
        pltpu.make_async_copy(ea_hbm.at[pl.ds(off, BLK)], attr_v, sem_a).start()

    def wait(idx_v, attr_v, sem_i, sem_a):
        pltpu.make_async_copy(dst_hbm.at[pl.ds(0, CH)], idx_v, sem_i).wait()
        pltpu.make_async_copy(ea_hbm.at[pl.ds(0, BLK)], attr_v, sem_a).wait()

    def process(idx_v, attr_v):
        def widen(i8, carry2):
            for u in range(8):
                i = i8 * 8 + u
                rows_v[i, pl.ds(0, EDGE_DIM)] = attr_v[i, :]
            return carry2

        lax.fori_loop(0, BLK // 8, widen, 0)
        pltpu.sync_copy(rows_v, s_acc.at[idx_v], add=True)

    start(0, idx0, attr0, si0, sa0)
    start(1, idx1, attr1, si1, sa1)
    plsc.subcore_barrier()

    def body(kk, carry):
        j0 = 2 * kk
        wait(idx0, attr0, si0, sa0)
        process(idx0, attr0)

        @pl.when(j0 + 2 < NB)
        def _():
            start(j0 + 2, idx0, attr0, si0, sa0)

        wait(idx1, attr1, si1, sa1)
        process(idx1, attr1)

        @pl.when(j0 + 3 < NB)
        def _():
            start(j0 + 3, idx1, attr1, si1, sa1)

        return carry

    lax.fori_loop(0, NB // 2, body, 0)

    toff = pl.multiple_of(base + NB * BLK, 8)
    pltpu.sync_copy(dst_hbm.at[pl.ds(toff, TAIL)], idxt)
    pltpu.sync_copy(ea_hbm.at[pl.ds(toff, TAIL)], attr0.at[pl.ds(0, TAIL)])

    def widen_t(i8, carry2):
        for u in range(8):
            i = i8 * 8 + u
            rows_v[i, pl.ds(0, EDGE_DIM)] = attr0[i, :]
        return carry2

    lax.fori_loop(0, TAIL // 8, widen_t, 0)
    pltpu.sync_copy(rows_v.at[pl.ds(0, TAIL)], s_acc.at[idxt], add=True)

    plsc.subcore_barrier()
    pltpu.sync_copy(s_acc.at[pl.ds(row0, STRIPE)], s_out.at[c, pl.ds(row0, STRIPE)])


@functools.partial(
    pl.kernel,
    mesh=_sc_mesh(),
    out_type=jax.ShapeDtypeStruct((NC, N_ACC, HIDDEN), jnp.float32),
    scratch_types=[
        pltpu.VMEM((CH,), jnp.int32),
        pltpu.VMEM((CH,), jnp.int32),
        pltpu.VMEM((TAIL,), jnp.int32),
        pltpu.VMEM((BLK, HIDDEN), jnp.float32),
        pltpu.VMEM((BLK, HIDDEN), jnp.float32),
        pltpu.VMEM_SHARED((N_ACC, HIDDEN), jnp.float32),
        pltpu.SemaphoreType.DMA,
        pltpu.SemaphoreType.DMA,
        pltpu.SemaphoreType.DMA,
        pltpu.SemaphoreType.DMA,
    ],
)
def _sc_scatter_msg(dst_hbm, msg_hbm, zs_hbm, n_out,
                    idx0, idx1, idxt, rows0, rows1, acc,
                    si0, si1, sr0, sr1):
    c = lax.axis_index("c")
    s = lax.axis_index("s")
    row0 = s * STRIPE
    pltpu.sync_copy(zs_hbm, acc.at[pl.ds(row0, STRIPE)])

    base = (c * NS + s) * EPC

    def start(blk, idx_v, rows_v, sem_i, sem_r):
        off = pl.multiple_of(base + blk * BLK, 8)
        pltpu.make_async_copy(dst_hbm.at[pl.ds(off, CH)], idx_v, sem_i).start()
        pltpu.make_async_copy(msg_hbm.at[pl.ds(off, BLK)], rows_v, sem_r).start()

    def wait(idx_v, rows_v, sem_i, sem_r):
        pltpu.make_async_copy(dst_hbm.at[pl.ds(0, CH)], idx_v, sem_i).wait()
        pltpu.make_async_copy(msg_hbm.at[pl.ds(0, BLK)], rows_v, sem_r).wait()

    def scatter(idx_v, rows_v):
        pltpu.sync_copy(rows_v, acc.at[idx_v], add=True)

    start(0, idx0, rows0, si0, sr0)
    start(1, idx1, rows1, si1, sr1)
    plsc.subcore_barrier()

    def body(kk, carry):
        j0 = 2 * kk
        wait(idx0, rows0, si0, sr0)
        scatter(idx0, rows0)

        @pl.when(j0 + 2 < NB)
        def _():
            start(j0 + 2, idx0, rows0, si0, sr0)

        wait(idx1, rows1, si1, sr1)
        scatter(idx1, rows1)

        @pl.when(j0 + 3 < NB)
        def _():
            start(j0 + 3, idx1, rows1, si1, sr1)

        return carry

    lax.fori_loop(0, NB // 2, body, 0)

    toff = pl.multiple_of(base + NB * BLK, 8)
    pltpu.sync_copy(dst_hbm.at[pl.ds(toff, TAIL)], idxt)
    pltpu.sync_copy(msg_hbm.at[pl.ds(toff, TAIL)], rows0.at[pl.ds(0, TAIL)])
    pltpu.sync_copy(rows0.at[pl.ds(0, TAIL)], acc.at[idxt], add=True)

    plsc.subcore_barrier()
    pltpu.sync_copy(acc.at[pl.ds(row0, STRIPE)], n_out.at[c, pl.ds(row0, STRIPE)])


def _mlp(x, w1, b1, w2, b2, w3, b3):
    h = jnp.maximum(jnp.dot(x, w1, preferred_element_type=jnp.float32) + b1, 0.0)
    h = jnp.maximum(jnp.dot(h, w2, preferred_element_type=jnp.float32) + b2, 0.0)
    return jnp.dot(h, w3, preferred_element_type=jnp.float32) + b3


def _edge_mlp_body(ea_ref, w1_ref, b1_ref, w2_ref, b2_ref, w3_ref, b3_ref, out_ref):
    out_ref[...] = _mlp(ea_ref[...], w1_ref[...], b1_ref[...], w2_ref[...],
                        b2_ref[...], w3_ref[...], b3_ref[...])


def _loop_mlp_body(st_ref, w1_ref, b1_ref, w2_ref, b2_ref, w3_ref, b3_ref,
                   out_ref):
    st = st_ref[0] + st_ref[1]
    ssum = st[:, 0:EDGE_DIM]
    deg = st[:, EDGE_DIM:EDGE_DIM + 1]
    ea = ssum / jnp.maximum(deg, 1.0)
    out_ref[...] = _mlp(ea, w1_ref[...], b1_ref[...], w2_ref[...],
                        b2_ref[...], w3_ref[...], b3_ref[...])


def _pool_head_body(p_ref, sm_ref, batch_ref, wc1_ref, bc1_ref, wc2_ref, bc2_ref,
                    out_ref, g_ref):
    i = pl.program_id(0)

    @pl.when(i == 0)
    def _():
        g_ref[...] = jnp.full((NUM_GRAPHS, HIDDEN), -jnp.inf, jnp.float32)

    x = p_ref[0] + p_ref[1] + sm_ref[...]
    b_col = batch_ref[...]
    bmin = b_col[0, 0]
    bmax = b_col[POOL_TILE - 1, 0]
    neg_inf = jnp.float32(-jnp.inf)
    for g in range(NUM_GRAPHS):
        @pl.when((bmin <= g) & (g <= bmax))
        def _(g=g):
            pen = jnp.where(b_col == g, 0.0, neg_inf)
            gmax = jnp.max(x + pen, axis=0, keepdims=True)
            g_ref[pl.ds(g, 1), :] = jnp.maximum(g_ref[pl.ds(g, 1), :], gmax)

    @pl.when(i == N_ACC // POOL_TILE - 1)
    def _():
        g = g_ref[...]
        gh = jnp.maximum(
            jnp.dot(g, wc1_ref[...], preferred_element_type=jnp.float32)
            + bc1_ref[...], 0.0)
        out_ref[...] = (jnp.dot(gh, wc2_ref[...], preferred_element_type=jnp.float32)
                        + bc2_ref[...])


def kernel(edge_index, edge_attr, batch, W1, b1, W2, b2, W3, b3, Wc1, bc1, Wc2, bc2):
    dst = edge_index[1].astype(jnp.int32)
    batch_pad = jnp.concatenate(
        [batch.astype(jnp.int32),
         jnp.full((N_ACC - N_NODES,), NUM_GRAPHS, jnp.int32)])
    batch_col = batch_pad.reshape(N_ACC, 1)

    init_rows = jnp.zeros((BLK, HIDDEN), jnp.float32).at[:, EDGE_DIM].set(1.0)
    zstripe = jnp.zeros((STRIPE, HIDDEN), jnp.float32)

    b1r = b1.reshape(1, HIDDEN)
    b2r = b2.reshape(1, HIDDEN)
    b3r = b3.reshape(1, HIDDEN)
    bc1r = bc1.reshape(1, HIDDEN)
    bc2r = bc2.reshape(1, NUM_CLASSES)

    stats = _sc_edge_stats(dst, edge_attr, init_rows, zstripe)

    msg = pl.pallas_call(
        _edge_mlp_body,
        grid=(N_EDGES // MLP_TILE,),
        in_specs=[
            pl.BlockSpec((MLP_TILE, EDGE_DIM), lambda i: (i, 0)),
            pl.BlockSpec((EDGE_DIM, HIDDEN), lambda i: (0, 0)),
            pl.BlockSpec((1, HIDDEN), lambda i: (0, 0)),
            pl.BlockSpec((HIDDEN, HIDDEN), lambda i: (0, 0)),
            pl.BlockSpec((1, HIDDEN), lambda i: (0, 0)),
            pl.BlockSpec((HIDDEN, HIDDEN), lambda i: (0, 0)),
            pl.BlockSpec((1, HIDDEN), lambda i: (0, 0)),
        ],
        out_specs=pl.BlockSpec((MLP_TILE, HIDDEN), lambda i: (i, 0)),
        out_shape=jax.ShapeDtypeStruct((N_EDGES, HIDDEN), jnp.float32),
    )(edge_attr, W1, b1r, W2, b2r, W3, b3r)

    self_msg = pl.pallas_call(
        _loop_mlp_body,
        grid=(N_ACC // LOOP_TILE,),
        in_specs=[
            pl.BlockSpec((NC, LOOP_TILE, HIDDEN), lambda i: (0, i, 0)),
            pl.BlockSpec((EDGE_DIM, HIDDEN), lambda i: (0, 0)),
            pl.BlockSpec((1, HIDDEN), lambda i: (0, 0)),
            pl.BlockSpec((HIDDEN, HIDDEN), lambda i: (0, 0)),
            pl.BlockSpec((1, HIDDEN), lambda i: (0, 0)),
            pl.BlockSpec((HIDDEN, HIDDEN), lambda i: (0, 0)),
            pl.BlockSpec((1, HIDDEN), lambda i: (0, 0)),
        ],
        out_specs=pl.BlockSpec((LOOP_TILE, HIDDEN), lambda i: (i, 0)),
        out_shape=jax.ShapeDtypeStruct((N_ACC, HIDDEN), jnp.float32),
    )(stats, W1, b1r, W2, b2r, W3, b3r)

    n_parts = _sc_scatter_msg(dst, msg, zstripe)

    logits = pl.pallas_call(
        _pool_head_body,
        grid=(N_ACC // POOL_TILE,),
        in_specs=[
            pl.BlockSpec((NC, POOL_TILE, HIDDEN), lambda i: (0, i, 0)),
            pl.BlockSpec((POOL_TILE, HIDDEN), lambda i: (i, 0)),
            pl.BlockSpec((POOL_TILE, 1), lambda i: (i, 0)),
            pl.BlockSpec((HIDDEN, HIDDEN), lambda i: (0, 0)),
            pl.BlockSpec((1, HIDDEN), lambda i: (0, 0)),
            pl.BlockSpec((HIDDEN, NUM_CLASSES), lambda i: (0, 0)),
            pl.BlockSpec((1, NUM_CLASSES), lambda i: (0, 0)),
        ],
        out_specs=pl.BlockSpec((NUM_GRAPHS, NUM_CLASSES), lambda i: (0, 0)),
        out_shape=jax.ShapeDtypeStruct((NUM_GRAPHS, NUM_CLASSES), jnp.float32),
        scratch_shapes=[pltpu.VMEM((NUM_GRAPHS, HIDDEN), jnp.float32)],
    )(n_parts, self_msg, batch_col, Wc1, bc1r, Wc2, bc2r)

    return logits


# speedup vs baseline: 1.1678x; 1.0424x over previous
import functools

import jax
import jax.numpy as jnp
from jax import lax
from jax.experimental import pallas as pl
from jax.experimental.pallas import tpu as pltpu
from jax.experimental.pallas import tpu_sc as plsc

N_NODES = 10000
N_EDGES = 640000
EDGE_DIM = 16
HIDDEN = 128
NUM_CLASSES = 10
NUM_GRAPHS = 64

NC = 2
NS = 16
NW = NC * NS

CH = 128
BLK = CH
EPC = N_EDGES // NW
NB = EPC // BLK
TAIL = EPC - NB * BLK
N_ACC = 10112
STRIPE = N_ACC // NS

MLP_TILE = 6400
LOOP_TILE = 1264
POOL_TILE = 1264


def _sc_mesh():
    return plsc.VectorSubcoreMesh(core_axis_name="c", subcore_axis_name="s")


@functools.partial(
    pl.kernel,
    mesh=_sc_mesh(),
    out_type=jax.ShapeDtypeStruct((NC, N_ACC, HIDDEN), jnp.float32),
    scratch_types=[
        pltpu.VMEM((CH,), jnp.int32),
        pltpu.VMEM((CH,), jnp.int32),
        pltpu.VMEM((TAIL,), jnp.int32),
        pltpu.VMEM((BLK, EDGE_DIM), jnp.float32),
        pltpu.VMEM((BLK, EDGE_DIM), jnp.float32),
        pltpu.VMEM((BLK, HIDDEN), jnp.float32),
        pltpu.VMEM_SHARED((N_ACC, HIDDEN), jnp.float32),
        pltpu.SemaphoreType.DMA,
        pltpu.SemaphoreType.DMA,
        pltpu.SemaphoreType.DMA,
        pltpu.SemaphoreType.DMA,
    ],
)
def _sc_edge_stats(dst_hbm, ea_hbm, init_hbm, zs_hbm, s_out,
                   idx0, idx1, idxt, attr0, attr1, rows_v, s_acc,
                   si0, si1, sa0, sa1):
    c = lax.axis_index("c")
    s = lax.axis_index("s")
    row0 = s * STRIPE
    pltpu.sync_copy(zs_hbm, s_acc.at[pl.ds(row0, STRIPE)])
    pltpu.sync_copy(init_hbm, rows_v)

    base = (c * NS + s) * EPC

    def start(blk, idx_v, attr_v, sem_i, sem_a):
        off = pl.multiple_of(base + blk * BLK, 8)
        pltpu.make_async_copy(dst_hbm.at[pl.ds(off, CH)], idx_v, sem_i).start()
        pltpu.make_async_copy(ea_hbm.at[pl.ds(off, BLK)], attr_v, sem_a).start()

    def wait(idx_v, attr_v, sem_i, sem_a):
        pltpu.make_async_copy(dst_hbm.at[pl.ds(0, CH)], idx_v, sem_i).wait()
        pltpu.make_async_copy(ea_hbm.at[pl.ds(0, BLK)], attr_v, sem_a).wait()

    def process(idx_v, attr_v):
        def widen(i8, carry2):
            for u in range(8):
                i = i8 * 8 + u
                rows_v[i, pl.ds(0, EDGE_DIM)] = attr_v[i, :]
            return carry2

        lax.fori_loop(0, BLK // 8, widen, 0)
        pltpu.sync_copy(rows_v, s_acc.at[idx_v], add=True)

    start(0, idx0, attr0, si0, sa0)
    start(1, idx1, attr1, si1, sa1)
    plsc.subcore_barrier()

    def body(kk, carry):
        j0 = 2 * kk
        wait(idx0, attr0, si0, sa0)
        process(idx0, attr0)

        @pl.when(j0 + 2 < NB)
        def _():
            start(j0 + 2, idx0, attr0, si0, sa0)

        wait(idx1, attr1, si1, sa1)
        process(idx1, attr1)

        @pl.when(j0 + 3 < NB)
        def _():
            start(j0 + 3, idx1, attr1, si1, sa1)

        return carry

    lax.fori_loop(0, NB // 2, body, 0)

    toff = pl.multiple_of(base + NB * BLK, 8)
    pltpu.sync_copy(dst_hbm.at[pl.ds(toff, TAIL)], idxt)
    pltpu.sync_copy(ea_hbm.at[pl.ds(toff, TAIL)], attr0.at[pl.ds(0, TAIL)])

    def widen_t(i8, carry2):
        for u in range(8):
            i = i8 * 8 + u
            rows_v[i, pl.ds(0, EDGE_DIM)] = attr0[i, :]
        return carry2

    lax.fori_loop(0, TAIL // 8, widen_t, 0)
    pltpu.sync_copy(rows_v.at[pl.ds(0, TAIL)], s_acc.at[idxt], add=True)

    plsc.subcore_barrier()
    pltpu.sync_copy(s_acc.at[pl.ds(row0, STRIPE)], s_out.at[c, pl.ds(row0, STRIPE)])


@functools.partial(
    pl.kernel,
    mesh=_sc_mesh(),
    out_type=jax.ShapeDtypeStruct((NC, N_ACC, HIDDEN), jnp.float32),
    scratch_types=[
        pltpu.VMEM((CH,), jnp.int32),
        pltpu.VMEM((CH,), jnp.int32),
        pltpu.VMEM((TAIL,), jnp.int32),
        pltpu.VMEM((BLK, HIDDEN), jnp.float32),
        pltpu.VMEM((BLK, HIDDEN), jnp.float32),
        pltpu.VMEM_SHARED((N_ACC, HIDDEN), jnp.float32),
        pltpu.SemaphoreType.DMA,
        pltpu.SemaphoreType.DMA,
        pltpu.SemaphoreType.DMA,
        pltpu.SemaphoreType.DMA,
    ],
)
def _sc_scatter_msg(dst_hbm, msg_hbm, zs_hbm, n_out,
                    idx0, idx1, idxt, rows0, rows1, acc,
                    si0, si1, sr0, sr1):
    c = lax.axis_index("c")
    s = lax.axis_index("s")
    row0 = s * STRIPE
    pltpu.sync_copy(zs_hbm, acc.at[pl.ds(row0, STRIPE)])

    base = (c * NS + s) * EPC

    def start(blk, idx_v, rows_v, sem_i, sem_r):
        off = pl.multiple_of(base + blk * BLK, 8)
        pltpu.make_async_copy(dst_hbm.at[pl.ds(off, CH)], idx_v, sem_i).start()
        pltpu.make_async_copy(msg_hbm.at[pl.ds(off, BLK)], rows_v, sem_r).start()

    def wait(idx_v, rows_v, sem_i, sem_r):
        pltpu.make_async_copy(dst_hbm.at[pl.ds(0, CH)], idx_v, sem_i).wait()
        pltpu.make_async_copy(msg_hbm.at[pl.ds(0, BLK)], rows_v, sem_r).wait()

    def scatter(idx_v, rows_v):
        pltpu.sync_copy(rows_v, acc.at[idx_v], add=True)

    start(0, idx0, rows0, si0, sr0)
    start(1, idx1, rows1, si1, sr1)
    plsc.subcore_barrier()

    def body(kk, carry):
        j0 = 2 * kk
        wait(idx0, rows0, si0, sr0)
        scatter(idx0, rows0)

        @pl.when(j0 + 2 < NB)
        def _():
            start(j0 + 2, idx0, rows0, si0, sr0)

        wait(idx1, rows1, si1, sr1)
        scatter(idx1, rows1)

        @pl.when(j0 + 3 < NB)
        def _():
            start(j0 + 3, idx1, rows1, si1, sr1)

        return carry

    lax.fori_loop(0, NB // 2, body, 0)

    toff = pl.multiple_of(base + NB * BLK, 8)
    pltpu.sync_copy(dst_hbm.at[pl.ds(toff, TAIL)], idxt)
    pltpu.sync_copy(msg_hbm.at[pl.ds(toff, TAIL)], rows0.at[pl.ds(0, TAIL)])
    pltpu.sync_copy(rows0.at[pl.ds(0, TAIL)], acc.at[idxt], add=True)

    plsc.subcore_barrier()
    pltpu.sync_copy(acc.at[pl.ds(row0, STRIPE)], n_out.at[c, pl.ds(row0, STRIPE)])


def _mlp(x, w1, b1, w2, b2, w3, b3):
    h = jnp.maximum(jnp.dot(x, w1, preferred_element_type=jnp.float32) + b1, 0.0)
    h = jnp.maximum(jnp.dot(h, w2, preferred_element_type=jnp.float32) + b2, 0.0)
    return jnp.dot(h, w3, preferred_element_type=jnp.float32) + b3


def _edge_mlp_body(ea_ref, w1_ref, b1_ref, w2_ref, b2_ref, w3_ref, b3_ref, out_ref):
    out_ref[...] = _mlp(ea_ref[...], w1_ref[...], b1_ref[...], w2_ref[...],
                        b2_ref[...], w3_ref[...], b3_ref[...])


def _loop_mlp_body(st_ref, w1_ref, b1_ref, w2_ref, b2_ref, w3_ref, b3_ref,
                   out_ref):
    st = st_ref[0] + st_ref[1]
    ssum = st[:, 0:EDGE_DIM]
    deg = st[:, EDGE_DIM:EDGE_DIM + 1]
    ea = ssum / jnp.maximum(deg, 1.0)
    out_ref[...] = _mlp(ea, w1_ref[...], b1_ref[...], w2_ref[...],
                        b2_ref[...], w3_ref[...], b3_ref[...])


def _pool_head_body(p_ref, sm_ref, batch_ref, wc1_ref, bc1_ref, wc2_ref, bc2_ref,
                    out_ref, g_ref):
    i = pl.program_id(0)

    @pl.when(i == 0)
    def _():
        g_ref[...] = jnp.full((NUM_GRAPHS, HIDDEN), -jnp.inf, jnp.float32)

    x = p_ref[0] + p_ref[1] + sm_ref[...]
    b_col = batch_ref[...]
    bmin = b_col[0, 0]
    bmax = b_col[POOL_TILE - 1, 0]
    neg_inf = jnp.float32(-jnp.inf)
    for g in range(NUM_GRAPHS):
        @pl.when((bmin <= g) & (g <= bmax))
        def _(g=g):
            pen = jnp.where(b_col == g, 0.0, neg_inf)
            gmax = jnp.max(x + pen, axis=0, keepdims=True)
            g_ref[pl.ds(g, 1), :] = jnp.maximum(g_ref[pl.ds(g, 1), :], gmax)

    @pl.when(i == N_ACC // POOL_TILE - 1)
    def _():
        g = g_ref[...]
        gh = jnp.maximum(
            jnp.dot(g, wc1_ref[...], preferred_element_type=jnp.float32)
            + bc1_ref[...], 0.0)
        out_ref[...] = (jnp.dot(gh, wc2_ref[...], preferred_element_type=jnp.float32)
                        + bc2_ref[...])


def kernel(edge_index, edge_attr, batch, W1, b1, W2, b2, W3, b3, Wc1, bc1, Wc2, bc2):
    dst = edge_index[1].astype(jnp.int32)
    batch_pad = jnp.concatenate(
        [batch.astype(jnp.int32),
         jnp.full((N_ACC - N_NODES,), NUM_GRAPHS, jnp.int32)])
    batch_col = batch_pad.reshape(N_ACC, 1)

    init_rows = jnp.zeros((BLK, HIDDEN), jnp.float32).at[:, EDGE_DIM].set(1.0)
    zstripe = jnp.zeros((STRIPE, HIDDEN), jnp.float32)

    b1r = b1.reshape(1, HIDDEN)
    b2r = b2.reshape(1, HIDDEN)
    b3r = b3.reshape(1, HIDDEN)
    bc1r = bc1.reshape(1, HIDDEN)
    bc2r = bc2.reshape(1, NUM_CLASSES)

    stats = _sc_edge_stats(dst, edge_attr, init_rows, zstripe)

    msg = pl.pallas_call(
        _edge_mlp_body,
        grid=(N_EDGES // MLP_TILE,),
        in_specs=[
            pl.BlockSpec((MLP_TILE, EDGE_DIM), lambda i: (i, 0)),
            pl.BlockSpec((EDGE_DIM, HIDDEN), lambda i: (0, 0)),
            pl.BlockSpec((1, HIDDEN), lambda i: (0, 0)),
            pl.BlockSpec((HIDDEN, HIDDEN), lambda i: (0, 0)),
            pl.BlockSpec((1, HIDDEN), lambda i: (0, 0)),
            pl.BlockSpec((HIDDEN, HIDDEN), lambda i: (0, 0)),
            pl.BlockSpec((1, HIDDEN), lambda i: (0, 0)),
        ],
        out_specs=pl.BlockSpec((MLP_TILE, HIDDEN), lambda i: (i, 0)),
        out_shape=jax.ShapeDtypeStruct((N_EDGES, HIDDEN), jnp.float32),
    )(edge_attr, W1, b1r, W2, b2r, W3, b3r)

    self_msg = pl.pallas_call(
        _loop_mlp_body,
        grid=(N_ACC // LOOP_TILE,),
        in_specs=[
            pl.BlockSpec((NC, LOOP_TILE, HIDDEN), lambda i: (0, i, 0)),
            pl.BlockSpec((EDGE_DIM, HIDDEN), lambda i: (0, 0)),
            pl.BlockSpec((1, HIDDEN), lambda i: (0, 0)),
            pl.BlockSpec((HIDDEN, HIDDEN), lambda i: (0, 0)),
            pl.BlockSpec((1, HIDDEN), lambda i: (0, 0)),
            pl.BlockSpec((HIDDEN, HIDDEN), lambda i: (0, 0)),
            pl.BlockSpec((1, HIDDEN), lambda i: (0, 0)),
        ],
        out_specs=pl.BlockSpec((LOOP_TILE, HIDDEN), lambda i: (i, 0)),
        out_shape=jax.ShapeDtypeStruct((N_ACC, HIDDEN), jnp.float32),
    )(stats, W1, b1r, W2, b2r, W3, b3r)

    n_parts = _sc_scatter_msg(dst, msg, zstripe)

    logits = pl.pallas_call(
        _pool_head_body,
        grid=(N_ACC // POOL_TILE,),
        in_specs=[
            pl.BlockSpec((NC, POOL_TILE, HIDDEN), lambda i: (0, i, 0)),
            pl.BlockSpec((POOL_TILE, HIDDEN), lambda i: (i, 0)),
            pl.BlockSpec((POOL_TILE, 1), lambda i: (i, 0)),
            pl.BlockSpec((HIDDEN, HIDDEN), lambda i: (0, 0)),
            pl.BlockSpec((1, HIDDEN), lambda i: (0, 0)),
            pl.BlockSpec((HIDDEN, NUM_CLASSES), lambda i: (0, 0)),
            pl.BlockSpec((1, NUM_CLASSES), lambda i: (0, 0)),
        ],
        out_specs=pl.BlockSpec((NUM_GRAPHS, NUM_CLASSES), lambda i: (0, 0)),
        out_shape=jax.ShapeDtypeStruct((NUM_GRAPHS, NUM_CLASSES), jnp.float32),
        scratch_shapes=[pltpu.VMEM((NUM_GRAPHS, HIDDEN), jnp.float32)],
    )(n_parts, self_msg, batch_col, Wc1, bc1r, Wc2, bc2r)

    return logits
